# baseline (device time: 16157 ns/iter reference)
import jax
import jax.numpy as jnp
from jax import lax
from jax.experimental import pallas as pl
from jax.experimental.pallas import tpu as pltpu

N_DEV = 4
B, Sq, Skv, Hq, Dh = 2, 256, 1024, 4, 64
HD = Hq * Dh
D = 512
S_SH = Skv // N_DEV
R = Sq + 8
F32 = jnp.float32
BF16 = jnp.bfloat16


def kernel(x, Wq, K_ext, V_ext, Wo):
    K2 = K_ext.reshape(B, S_SH, HD)
    V2 = V_ext.reshape(B, S_SH, HD)

    def body(x_ref, wq_ref, k_ref, v_ref, wo_ref, out_ref,
             pbuf, rbuf, csend, crecv):
        my = lax.axis_index("i")
        left = lax.rem(my + N_DEV - 1, N_DEV)
        right = lax.rem(my + 1, N_DEV)
        diag = lax.rem(my + 2, N_DEV)

        barrier = pltpu.get_barrier_semaphore()
        for nbr in (left, right, diag):
            pl.semaphore_signal(barrier, inc=1, device_id=(nbr,),
                                device_id_type=pl.DeviceIdType.MESH)
        pl.semaphore_wait(barrier, 3)

        koff = my * S_SH
        qi = lax.broadcasted_iota(jnp.int32, (Sq, S_SH), 0)
        kig = lax.broadcasted_iota(jnp.int32, (Sq, S_SH), 1) + koff
        mask = (jnp.abs(qi - kig) <= 128) | (kig < 32) | (qi < 32)

        def rdma(slot, b, target):
            return pltpu.make_async_remote_copy(
                src_ref=pbuf.at[b], dst_ref=rbuf.at[slot, b],
                send_sem=csend.at[slot, b], recv_sem=crecv.at[slot, b],
                device_id=(target,), device_id_type=pl.DeviceIdType.MESH)

        sends = [[None] * 3 for _ in range(B)]
        for b in range(B):
            q_b = jnp.dot(x_ref[b], wq_ref[...],
                          preferred_element_type=F32)
            lcols = []
            for h in range(Hq):
                qh = q_b[:, h * Dh:(h + 1) * Dh]
                kh = k_ref[b, :, h * Dh:(h + 1) * Dh]
                s = lax.dot_general(
                    qh, kh, (((1,), (1,)), ((), ())),
                    preferred_element_type=F32) * 0.125
                w = jnp.where(mask, jnp.exp(s), 0.0)
                vh = v_ref[b, :, h * Dh:(h + 1) * Dh]
                pbuf[b, :Sq, h * Dh:(h + 1) * Dh] = jnp.dot(
                    w, vh, preferred_element_type=F32).astype(BF16)
                lcols.append(jnp.sum(w, axis=1, keepdims=True))
            l_b = jnp.concatenate(
                lcols + [jnp.zeros((Sq, 8 - Hq), F32)], axis=1)
            pbuf[b, Sq:, :] = jnp.transpose(l_b).astype(BF16)
            for slot, target in ((0, right), (1, left), (2, diag)):
                sends[b][slot] = rdma(slot, b, target)
                sends[b][slot].start()

        for b in range(B):
            for slot in range(3):
                sends[b][slot].wait()
            tot = (pbuf[b].astype(F32) + rbuf[0, b].astype(F32)
                   + rbuf[1, b].astype(F32) + rbuf[2, b].astype(F32))
            ctx = tot[:Sq, :]
            l_b = jnp.transpose(tot[Sq:, :])
            parts = []
            for h in range(Hq):
                parts.append(ctx[:, h * Dh:(h + 1) * Dh]
                             / l_b[:, h:h + 1])
            ctx_n = jnp.concatenate(parts, axis=1)
            out_ref[b] = jnp.dot(ctx_n, wo_ref[...],
                                 preferred_element_type=F32)

    return pl.pallas_call(
        body,
        out_shape=jax.ShapeDtypeStruct((B, Sq, D), jnp.float32),
        in_specs=[pl.BlockSpec(memory_space=pltpu.VMEM)] * 5,
        out_specs=pl.BlockSpec(memory_space=pltpu.VMEM),
        scratch_shapes=[
            pltpu.VMEM((B, R, HD), BF16),
            pltpu.VMEM((3, B, R, HD), BF16),
            pltpu.SemaphoreType.DMA((3, B)),
            pltpu.SemaphoreType.DMA((3, B)),
        ],
        compiler_params=pltpu.CompilerParams(collective_id=0),
    )(x, Wq, K2, V2, Wo)


# device time: 6935 ns/iter; 2.3298x vs baseline; 2.3298x over previous
import jax
import jax.numpy as jnp
from jax import lax
from jax.experimental import pallas as pl
from jax.experimental.pallas import tpu as pltpu

N_DEV = 4
B, Sq, Skv, Hq, Dh = 2, 256, 1024, 4, 64
HD = Hq * Dh
D = 512
S_SH = Skv // N_DEV
R = Sq + 8
F32 = jnp.float32
BF16 = jnp.bfloat16


def kernel(x, Wq, K_ext, V_ext, Wo):
    K2 = K_ext.reshape(B, S_SH, HD)
    V2 = V_ext.reshape(B, S_SH, HD)

    def body(x_ref, wq_ref, k_ref, v_ref, wo_ref, out_ref,
             pbuf, rbuf, csend, crecv):
        my = lax.axis_index("i")
        left = lax.rem(my + N_DEV - 1, N_DEV)
        right = lax.rem(my + 1, N_DEV)
        diag = lax.rem(my + 2, N_DEV)

        barrier = pltpu.get_barrier_semaphore()
        for nbr in (left, right, diag):
            pl.semaphore_signal(barrier, inc=1, device_id=(nbr,),
                                device_id_type=pl.DeviceIdType.MESH)
        pl.semaphore_wait(barrier, 3)

        koff = my * S_SH
        qi = lax.broadcasted_iota(jnp.int32, (Sq, S_SH), 0)
        kig = lax.broadcasted_iota(jnp.int32, (Sq, S_SH), 1) + koff
        mask = (jnp.abs(qi - kig) <= 128) | (kig < 32) | (qi < 32)

        def rdma(slot, b, target):
            return pltpu.make_async_remote_copy(
                src_ref=pbuf.at[b], dst_ref=rbuf.at[slot, b],
                send_sem=csend.at[slot, b], recv_sem=crecv.at[slot, b],
                device_id=(target,), device_id_type=pl.DeviceIdType.MESH)

        sends = [[None] * 3 for _ in range(B)]
        wq16 = wq_ref[...].astype(BF16)
        wo16 = wo_ref[...].astype(BF16)
        for b in range(B):
            q_b = jnp.dot(x_ref[b].astype(BF16), wq16,
                          preferred_element_type=F32).astype(BF16)
            k16 = k_ref[b].astype(BF16)
            v16 = v_ref[b].astype(BF16)
            lcols = []
            for h in range(Hq):
                qh = q_b[:, h * Dh:(h + 1) * Dh]
                kh = k16[:, h * Dh:(h + 1) * Dh]
                s = lax.dot_general(
                    qh, kh, (((1,), (1,)), ((), ())),
                    preferred_element_type=F32) * 0.125
                w = jnp.where(mask, jnp.exp(s), 0.0)
                vh = v16[:, h * Dh:(h + 1) * Dh]
                pbuf[b, :Sq, h * Dh:(h + 1) * Dh] = jnp.dot(
                    w.astype(BF16), vh,
                    preferred_element_type=F32).astype(BF16)
                lcols.append(jnp.sum(w, axis=1, keepdims=True))
            l_b = jnp.concatenate(
                lcols + [jnp.zeros((Sq, 8 - Hq), F32)], axis=1)
            pbuf[b, Sq:, :] = jnp.transpose(l_b).astype(BF16)
            for slot, target in ((0, right), (1, left), (2, diag)):
                sends[b][slot] = rdma(slot, b, target)
                sends[b][slot].start()

        for b in range(B):
            for slot in range(3):
                sends[b][slot].wait()
            tot = (pbuf[b].astype(F32) + rbuf[0, b].astype(F32)
                   + rbuf[1, b].astype(F32) + rbuf[2, b].astype(F32))
            ctx = tot[:Sq, :]
            l_b = jnp.transpose(tot[Sq:, :])
            parts = []
            for h in range(Hq):
                parts.append(ctx[:, h * Dh:(h + 1) * Dh]
                             / l_b[:, h:h + 1])
            ctx_n = jnp.concatenate(parts, axis=1)
            out_ref[b] = jnp.dot(ctx_n.astype(BF16), wo16,
                                 preferred_element_type=F32)

    return pl.pallas_call(
        body,
        out_shape=jax.ShapeDtypeStruct((B, Sq, D), jnp.float32),
        in_specs=[pl.BlockSpec(memory_space=pltpu.VMEM)] * 5,
        out_specs=pl.BlockSpec(memory_space=pltpu.VMEM),
        scratch_shapes=[
            pltpu.VMEM((B, R, HD), BF16),
            pltpu.VMEM((3, B, R, HD), BF16),
            pltpu.SemaphoreType.DMA((3, B)),
            pltpu.SemaphoreType.DMA((3, B)),
        ],
        compiler_params=pltpu.CompilerParams(collective_id=0),
    )(x, Wq, K2, V2, Wo)
